# Initial kernel scaffold; baseline (speedup 1.0000x reference)
import jax
import jax.numpy as jnp
from jax.experimental import pallas as pl


def kernel(x, pos, batch, Wo1, bo1, g1, be1, Wo2, bo2, g2, be2, Wo3, bo3, W1, b1, W2, b2):
    raise NotImplementedError("write your pallas kernel here")


# speedup vs baseline: 2.8442x; 2.8442x over previous
import functools

import jax
import jax.numpy as jnp
import numpy as np
from jax import lax
from jax.experimental import pallas as pl
from jax.experimental.pallas import tpu as pltpu

_R2 = 0.2 * 0.2
_K = 64
_EPS = 1e-5

_NPTS = 10000
_NROW = 80
_NPAD = _NROW * 128
_M = 2500
_BQ = 8
_MPAD = 2504
_EPAD = _MPAD * _K


def _fps_body(px_ref, py_ref, pz_ref, idx_ref):
    px = px_ref[...]
    py = py_ref[...]
    pz = pz_ref[...]
    rows = lax.broadcasted_iota(jnp.int32, (_NROW, 128), 0)
    cols = lax.broadcasted_iota(jnp.int32, (_NROW, 128), 1)
    iota = rows * 128 + cols
    padm = iota >= _NPTS
    idx_ref[0] = 0

    def dist_to(n):
        msk = iota == n
        x0 = jnp.sum(jnp.where(msk, px, 0.0))
        y0 = jnp.sum(jnp.where(msk, py, 0.0))
        z0 = jnp.sum(jnp.where(msk, pz, 0.0))
        dx = px - x0
        dy = py - y0
        dz = pz - z0
        return dx * dx + dy * dy + dz * dz

    mind = jnp.where(padm, -jnp.inf, dist_to(0))

    def body(i, mind):
        m = jnp.max(mind)
        cand = jnp.where(mind == m, iota, jnp.int32(2**30))
        nxt = jnp.min(cand)
        idx_ref[i] = nxt
        return jnp.minimum(mind, dist_to(nxt))

    lax.fori_loop(1, _M, body, mind)


def _fps(px, py, pz, interpret=False):
    return pl.pallas_call(
        _fps_body,
        out_shape=jax.ShapeDtypeStruct((_M,), jnp.int32),
        out_specs=pl.BlockSpec(memory_space=pltpu.SMEM),
        interpret=interpret,
    )(px, py, pz)


def _xw_body(x_ref, w_ref, o_ref):
    o_ref[...] = jnp.dot(x_ref[...], w_ref[...],
                         preferred_element_type=jnp.float32)


def _xw(xpad, w1x, interpret=False):
    return pl.pallas_call(
        _xw_body,
        grid=(10,),
        in_specs=[pl.BlockSpec((1024, 128), lambda i: (i, 0)),
                  pl.BlockSpec((128, 128), lambda i: (0, 0))],
        out_specs=pl.BlockSpec((1024, 128), lambda i: (i, 0)),
        out_shape=jax.ShapeDtypeStruct((_NPAD, 128), jnp.float32),
        interpret=interpret,
    )(xpad, w1x)


def _conv_body(xj_ref, rv_ref, w1r_ref, b1_ref, w2_ref, b2_ref, o_ref):
    xj = xj_ref[...]
    rv = rv_ref[...]
    h1 = xj + jnp.dot(rv, w1r_ref[...],
                      preferred_element_type=jnp.float32) + b1_ref[...]
    h1 = jnp.maximum(h1, 0.0)
    h2 = jnp.dot(h1, w2_ref[...],
                 preferred_element_type=jnp.float32) + b2_ref[...]
    h2 = jnp.maximum(h2, 0.0)
    validc = rv[:, 3:4] > 0.5
    h2m = jnp.where(validc, h2, -jnp.inf)
    mx = jnp.max(h2m.reshape(_BQ, _K, 128), axis=1)
    o_ref[...] = jnp.where(jnp.isfinite(mx), mx, 0.0)


def _conv(xj, relv, w1r8, b1, w2, b2, interpret=False):
    nblk = _MPAD // _BQ
    return pl.pallas_call(
        _conv_body,
        grid=(nblk,),
        in_specs=[pl.BlockSpec((_BQ * _K, 128), lambda c: (c, 0)),
                  pl.BlockSpec((_BQ * _K, 8), lambda c: (c, 0)),
                  pl.BlockSpec((8, 128), lambda c: (0, 0)),
                  pl.BlockSpec((1, 128), lambda c: (0, 0)),
                  pl.BlockSpec((128, 128), lambda c: (0, 0)),
                  pl.BlockSpec((1, 128), lambda c: (0, 0))],
        out_specs=pl.BlockSpec((_BQ, 128), lambda c: (c, 0)),
        out_shape=jax.ShapeDtypeStruct((_MPAD, 128), jnp.float32),
        interpret=interpret,
    )(xj, relv, w1r8, b1.reshape(1, 128), w2, b2.reshape(1, 128))


def kernel(x, pos, batch, Wo1, bo1, g1, be1, Wo2, bo2, g2, be2, Wo3, bo3,
           W1, b1, W2, b2):
    h = pos @ Wo1 + bo1
    h = h / jnp.sqrt(1.0 + _EPS) * g1 + be1
    h = jax.nn.relu(h)
    h = h @ Wo2 + bo2
    h = h / jnp.sqrt(1.0 + _EPS) * g2 + be2
    h = jax.nn.relu(h)
    off = h @ Wo3 + bo3
    dpos = pos + jnp.tanh(off) * 0.1

    padv = jnp.full((_NPAD - _NPTS,), 1e9, jnp.float32)
    px = jnp.concatenate([dpos[:, 0], padv]).reshape(_NROW, 128)
    py = jnp.concatenate([dpos[:, 1], padv]).reshape(_NROW, 128)
    pz = jnp.concatenate([dpos[:, 2], padv]).reshape(_NROW, 128)

    idx = _fps(px, py, pz)
    q = dpos[idx]

    d2 = jnp.sum((q[:, None, :] - dpos[None, :, :]) ** 2, axis=-1)
    negd, nbr = lax.top_k(-d2, _K)
    valid = (-negd) <= _R2

    xpad = jnp.pad(x, ((0, _NPAD - _NPTS), (0, 0)))
    xw = _xw(xpad, W1[:128, :])

    nbrf = nbr.reshape(-1)
    xj = xw[nbrf]
    relq = dpos[nbrf] - jnp.repeat(q, _K, axis=0)
    vf = valid.reshape(-1, 1).astype(jnp.float32)
    relv = jnp.concatenate(
        [relq, vf, jnp.zeros((_M * _K, 4), jnp.float32)], axis=1)
    xj = jnp.pad(xj, ((0, _EPAD - _M * _K), (0, 0)))
    relv = jnp.pad(relv, ((0, _EPAD - _M * _K), (0, 0)))

    w1r8 = jnp.pad(W1[128:, :], ((0, 5), (0, 0)))

    out = _conv(xj, relv, w1r8, b1, W2, b2)[:_M]
    return (out, q, batch[idx])


# speedup vs baseline: 8.7268x; 3.0683x over previous
import functools

import jax
import jax.numpy as jnp
import numpy as np
from jax import lax
from jax.experimental import pallas as pl
from jax.experimental.pallas import tpu as pltpu
from jax.experimental.pallas import tpu_sc as plsc

_R2 = 0.2 * 0.2
_K = 64
_EPS = 1e-5

_NPTS = 10000
_NROW = 80
_NPAD = _NROW * 128
_M = 2500
_BQ = 8
_MPAD = 2504
_EPAD = _MPAD * _K


def _fps_body(px_ref, py_ref, pz_ref, idx_ref):
    px = px_ref[...]
    py = py_ref[...]
    pz = pz_ref[...]
    rows = lax.broadcasted_iota(jnp.int32, (_NROW, 128), 0)
    cols = lax.broadcasted_iota(jnp.int32, (_NROW, 128), 1)
    iota = rows * 128 + cols
    padm = iota >= _NPTS
    idx_ref[0] = 0

    def dist_to(n):
        msk = iota == n
        x0 = jnp.sum(jnp.where(msk, px, 0.0))
        y0 = jnp.sum(jnp.where(msk, py, 0.0))
        z0 = jnp.sum(jnp.where(msk, pz, 0.0))
        dx = px - x0
        dy = py - y0
        dz = pz - z0
        return dx * dx + dy * dy + dz * dz

    mind = jnp.where(padm, -jnp.inf, dist_to(0))

    def body(i, mind):
        m = jnp.max(mind)
        cand = jnp.where(mind == m, iota, jnp.int32(2**30))
        nxt = jnp.min(cand)
        idx_ref[i] = nxt
        return jnp.minimum(mind, dist_to(nxt))

    lax.fori_loop(1, _M, body, mind)


def _fps(px, py, pz, interpret=False):
    return pl.pallas_call(
        _fps_body,
        out_shape=jax.ShapeDtypeStruct((_M,), jnp.int32),
        out_specs=pl.BlockSpec(memory_space=pltpu.SMEM),
        interpret=interpret,
    )(px, py, pz)


_SC_NC = 2
_SC_NW = 32
_CPW = 79
_QPAD = _SC_NW * _CPW
_P16 = 10016
_NCH1 = _P16 // 16
_CAP = 1024
_SLOTS = _CPW * _K
_R2BITS = int(np.array(_R2, np.float32).view(np.int32))


def _bq_body(px_hbm, py_hbm, pz_hbm, qx_hbm, qy_hbm, qz_hbm,
             nbr_hbm, val_hbm,
             px_v, py_v, pz_v, qx_v, qy_v, qz_v,
             cd2_v, cix_v, nbr_v, val_v):
    wid = lax.axis_index("s") * _SC_NC + lax.axis_index("c")
    pltpu.sync_copy(px_hbm, px_v)
    pltpu.sync_copy(py_hbm, py_v)
    pltpu.sync_copy(pz_hbm, pz_v)
    pltpu.sync_copy(qx_hbm, qx_v)
    pltpu.sync_copy(qy_hbm, qy_v)
    pltpu.sync_copy(qz_hbm, qz_v)

    lane = lax.broadcasted_iota(jnp.int32, (16,), 0)
    zero16i = jnp.zeros((16,), jnp.int32)
    zero16f = jnp.zeros((16,), jnp.float32)
    one16f = jnp.ones((16,), jnp.float32)
    r2v = jnp.full((16,), _R2, jnp.float32)

    def initb(j, carry):
        sl = pl.ds(j * 16, 16)
        nbr_v[sl] = zero16i
        val_v[sl] = zero16f
        return carry

    lax.fori_loop(0, _SLOTS // 16, initb, 0)

    def per_centroid(i, carry):
        c = wid * _CPW + i
        cvec = zero16i + c
        qxv = plsc.load_gather(qx_v, [cvec])
        qyv = plsc.load_gather(qy_v, [cvec])
        qzv = plsc.load_gather(qz_v, [cvec])

        def chunk1(j, base):
            sl = pl.ds(j * 16, 16)
            dx = px_v[sl] - qxv
            dy = py_v[sl] - qyv
            dz = pz_v[sl] - qzv
            d2 = dx * dx + dy * dy + dz * dz
            msk = d2 <= r2v
            inc = plsc.cumsum(msk.astype(jnp.int32))
            pos = base + inc - 1
            wm = msk & (pos < _CAP)
            posc = jnp.where(wm, pos, 0)
            plsc.store_scatter(cd2_v, [posc], d2, mask=wm)
            plsc.store_scatter(cix_v, [posc], j * 16 + lane, mask=wm)
            return base + jnp.max(inc)

        cnt = lax.fori_loop(0, _NCH1, chunk1, jnp.int32(0))
        cnt = jnp.minimum(cnt, jnp.int32(_CAP))
        nch = lax.div(cnt + 15, jnp.int32(16))

        def count_le(vbits):
            vv = zero16i + vbits

            def cc(j, acc):
                sl = pl.ds(j * 16, 16)
                b = plsc.bitcast(cd2_v[sl], jnp.int32)
                ok = ((j * 16 + lane) < cnt) & (b <= vv)
                return acc + jnp.max(plsc.cumsum(ok.astype(jnp.int32)))

            return lax.fori_loop(0, nch, cc, jnp.int32(0))

        def bis(_, lh):
            lo, hi = lh
            mid = lax.div(lo + hi, jnp.int32(2))
            ge = count_le(mid) >= _K
            return (jnp.where(ge, lo, mid + 1), jnp.where(ge, mid, hi))

        _, hi = lax.fori_loop(0, 31, bis,
                              (jnp.int32(0), jnp.int32(_R2BITS)))
        thr = jnp.where(cnt > _K, hi, jnp.int32(_R2BITS))
        thrv = zero16i + thr

        def chunk2(j, base):
            sl = pl.ds(j * 16, 16)
            b = plsc.bitcast(cd2_v[sl], jnp.int32)
            ix = cix_v[sl]
            sel = ((j * 16 + lane) < cnt) & (b <= thrv)
            inc = plsc.cumsum(sel.astype(jnp.int32))
            slot = base + inc - 1
            wm = sel & (slot < _K)
            slotc = jnp.where(wm, i * _K + slot, 0)
            plsc.store_scatter(nbr_v, [slotc], ix, mask=wm)
            plsc.store_scatter(val_v, [slotc], one16f, mask=wm)
            return base + jnp.max(inc)

        lax.fori_loop(0, nch, chunk2, jnp.int32(0))
        return carry

    lax.fori_loop(0, _CPW, per_centroid, 0)
    pltpu.sync_copy(nbr_v, nbr_hbm.at[wid])
    pltpu.sync_copy(val_v, val_hbm.at[wid])


def _ballquery(p16, q16, interpret=False):
    mesh = plsc.VectorSubcoreMesh(core_axis_name="c", subcore_axis_name="s")
    f = pl.kernel(
        _bq_body,
        out_type=(jax.ShapeDtypeStruct((_SC_NW, _SLOTS), jnp.int32),
                  jax.ShapeDtypeStruct((_SC_NW, _SLOTS), jnp.float32)),
        mesh=mesh,
        scratch_types=[pltpu.VMEM((_P16,), jnp.float32)] * 3
        + [pltpu.VMEM((_QPAD,), jnp.float32)] * 3
        + [pltpu.VMEM((_CAP,), jnp.float32), pltpu.VMEM((_CAP,), jnp.int32),
           pltpu.VMEM((_SLOTS,), jnp.int32), pltpu.VMEM((_SLOTS,), jnp.float32)],
        compiler_params=pltpu.CompilerParams(needs_layout_passes=False),
        interpret=interpret,
    )
    return f(p16[0], p16[1], p16[2], q16[0], q16[1], q16[2])


def _xw_body(x_ref, w_ref, o_ref):
    o_ref[...] = jnp.dot(x_ref[...], w_ref[...],
                         preferred_element_type=jnp.float32)


def _xw(xpad, w1x, interpret=False):
    return pl.pallas_call(
        _xw_body,
        grid=(10,),
        in_specs=[pl.BlockSpec((1024, 128), lambda i: (i, 0)),
                  pl.BlockSpec((128, 128), lambda i: (0, 0))],
        out_specs=pl.BlockSpec((1024, 128), lambda i: (i, 0)),
        out_shape=jax.ShapeDtypeStruct((_NPAD, 128), jnp.float32),
        interpret=interpret,
    )(xpad, w1x)


def _conv_body(xj_ref, rv_ref, w1r_ref, b1_ref, w2_ref, b2_ref, o_ref):
    xj = xj_ref[...]
    rv = rv_ref[...]
    h1 = xj + jnp.dot(rv, w1r_ref[...],
                      preferred_element_type=jnp.float32) + b1_ref[...]
    h1 = jnp.maximum(h1, 0.0)
    h2 = jnp.dot(h1, w2_ref[...],
                 preferred_element_type=jnp.float32) + b2_ref[...]
    h2 = jnp.maximum(h2, 0.0)
    validc = rv[:, 3:4] > 0.5
    h2m = jnp.where(validc, h2, -jnp.inf)
    mx = jnp.max(h2m.reshape(_BQ, _K, 128), axis=1)
    o_ref[...] = jnp.where(jnp.isfinite(mx), mx, 0.0)


def _conv(xj, relv, w1r8, b1, w2, b2, interpret=False):
    nblk = _MPAD // _BQ
    return pl.pallas_call(
        _conv_body,
        grid=(nblk,),
        in_specs=[pl.BlockSpec((_BQ * _K, 128), lambda c: (c, 0)),
                  pl.BlockSpec((_BQ * _K, 8), lambda c: (c, 0)),
                  pl.BlockSpec((8, 128), lambda c: (0, 0)),
                  pl.BlockSpec((1, 128), lambda c: (0, 0)),
                  pl.BlockSpec((128, 128), lambda c: (0, 0)),
                  pl.BlockSpec((1, 128), lambda c: (0, 0))],
        out_specs=pl.BlockSpec((_BQ, 128), lambda c: (c, 0)),
        out_shape=jax.ShapeDtypeStruct((_MPAD, 128), jnp.float32),
        interpret=interpret,
    )(xj, relv, w1r8, b1.reshape(1, 128), w2, b2.reshape(1, 128))


def kernel(x, pos, batch, Wo1, bo1, g1, be1, Wo2, bo2, g2, be2, Wo3, bo3,
           W1, b1, W2, b2):
    h = pos @ Wo1 + bo1
    h = h / jnp.sqrt(1.0 + _EPS) * g1 + be1
    h = jax.nn.relu(h)
    h = h @ Wo2 + bo2
    h = h / jnp.sqrt(1.0 + _EPS) * g2 + be2
    h = jax.nn.relu(h)
    off = h @ Wo3 + bo3
    dpos = pos + jnp.tanh(off) * 0.1

    padv = jnp.full((_NPAD - _NPTS,), 1e9, jnp.float32)
    px = jnp.concatenate([dpos[:, 0], padv]).reshape(_NROW, 128)
    py = jnp.concatenate([dpos[:, 1], padv]).reshape(_NROW, 128)
    pz = jnp.concatenate([dpos[:, 2], padv]).reshape(_NROW, 128)

    idx = _fps(px, py, pz)
    q = dpos[idx]

    padp = jnp.full((_P16 - _NPTS,), 1e9, jnp.float32)
    p16 = [jnp.concatenate([dpos[:, k], padp]) for k in range(3)]
    padq = jnp.full((_QPAD - _M,), 1e9, jnp.float32)
    q16 = [jnp.concatenate([q[:, k], padq]) for k in range(3)]
    nbr32, val32 = _ballquery(p16, q16)
    nbr = nbr32.reshape(_QPAD, _K)[:_M]
    valid = val32.reshape(_QPAD, _K)[:_M] > 0.5

    xpad = jnp.pad(x, ((0, _NPAD - _NPTS), (0, 0)))
    xw = _xw(xpad, W1[:128, :])

    nbrf = nbr.reshape(-1)
    xj = xw[nbrf]
    relq = dpos[nbrf] - jnp.repeat(q, _K, axis=0)
    vf = valid.reshape(-1, 1).astype(jnp.float32)
    relv = jnp.concatenate(
        [relq, vf, jnp.zeros((_M * _K, 4), jnp.float32)], axis=1)
    xj = jnp.pad(xj, ((0, _EPAD - _M * _K), (0, 0)))
    relv = jnp.pad(relv, ((0, _EPAD - _M * _K), (0, 0)))

    w1r8 = jnp.pad(W1[128:, :], ((0, 5), (0, 0)))

    out = _conv(xj, relv, w1r8, b1, W2, b2)[:_M]
    return (out, q, batch[idx])


# speedup vs baseline: 9.0913x; 1.0418x over previous
import functools

import jax
import jax.numpy as jnp
import numpy as np
from jax import lax
from jax.experimental import pallas as pl
from jax.experimental.pallas import tpu as pltpu
from jax.experimental.pallas import tpu_sc as plsc

_R2 = 0.2 * 0.2
_K = 64
_EPS = 1e-5

_NPTS = 10000
_NROW = 80
_NPAD = _NROW * 128
_M = 2500
_BQ = 8
_MPAD = 2560
_EPAD = _MPAD * _K


def _fps_body(px_ref, py_ref, pz_ref, idx_ref):
    px = px_ref[...]
    py = py_ref[...]
    pz = pz_ref[...]
    rows = lax.broadcasted_iota(jnp.int32, (_NROW, 128), 0)
    cols = lax.broadcasted_iota(jnp.int32, (_NROW, 128), 1)
    iota = rows * 128 + cols
    padm = iota >= _NPTS
    idx_ref[0] = 0

    def dist_to(n):
        msk = iota == n
        x0 = jnp.sum(jnp.where(msk, px, 0.0))
        y0 = jnp.sum(jnp.where(msk, py, 0.0))
        z0 = jnp.sum(jnp.where(msk, pz, 0.0))
        dx = px - x0
        dy = py - y0
        dz = pz - z0
        return dx * dx + dy * dy + dz * dz

    mind = jnp.where(padm, -jnp.inf, dist_to(0))

    def body(i, mind):
        m = jnp.max(mind)
        cand = jnp.where(mind == m, iota, jnp.int32(2**30))
        nxt = jnp.min(cand)
        idx_ref[i] = nxt
        return jnp.minimum(mind, dist_to(nxt))

    lax.fori_loop(1, _M, body, mind)


def _fps(px, py, pz, interpret=False):
    return pl.pallas_call(
        _fps_body,
        out_shape=jax.ShapeDtypeStruct((_M,), jnp.int32),
        out_specs=pl.BlockSpec(memory_space=pltpu.SMEM),
        interpret=interpret,
    )(px, py, pz)


_SC_NC = 2
_SC_NW = 32
_CPW = 79
_QPAD = _SC_NW * _CPW
_P16 = 10016
_NCH1 = _P16 // 16
_CAP = 1024
_SLOTS = _CPW * _K
_R2BITS = int(np.array(_R2, np.float32).view(np.int32))


def _bq_body(px_hbm, py_hbm, pz_hbm, qx_hbm, qy_hbm, qz_hbm,
             nbr_hbm, val_hbm,
             px_v, py_v, pz_v, qx_v, qy_v, qz_v,
             cd2_v, cix_v, nbr_v, val_v):
    wid = lax.axis_index("s") * _SC_NC + lax.axis_index("c")
    pltpu.sync_copy(px_hbm, px_v)
    pltpu.sync_copy(py_hbm, py_v)
    pltpu.sync_copy(pz_hbm, pz_v)
    pltpu.sync_copy(qx_hbm, qx_v)
    pltpu.sync_copy(qy_hbm, qy_v)
    pltpu.sync_copy(qz_hbm, qz_v)

    lane = lax.broadcasted_iota(jnp.int32, (16,), 0)
    zero16i = jnp.zeros((16,), jnp.int32)
    zero16f = jnp.zeros((16,), jnp.float32)
    one16f = jnp.ones((16,), jnp.float32)
    r2v = jnp.full((16,), _R2, jnp.float32)

    def initb(j, carry):
        sl = pl.ds(j * 16, 16)
        nbr_v[sl] = zero16i
        val_v[sl] = zero16f
        return carry

    lax.fori_loop(0, _SLOTS // 16, initb, 0)

    def per_centroid(i, carry):
        c = wid * _CPW + i
        cvec = zero16i + c
        qxv = plsc.load_gather(qx_v, [cvec])
        qyv = plsc.load_gather(qy_v, [cvec])
        qzv = plsc.load_gather(qz_v, [cvec])

        def chunk1(j, base):
            sl = pl.ds(j * 16, 16)
            dx = px_v[sl] - qxv
            dy = py_v[sl] - qyv
            dz = pz_v[sl] - qzv
            d2 = dx * dx + dy * dy + dz * dz
            msk = d2 <= r2v
            inc = plsc.cumsum(msk.astype(jnp.int32))
            pos = base + inc - 1
            wm = msk & (pos < _CAP)
            posc = jnp.where(wm, pos, 0)
            plsc.store_scatter(cd2_v, [posc], d2, mask=wm)
            plsc.store_scatter(cix_v, [posc], j * 16 + lane, mask=wm)
            return base + jnp.max(inc)

        cnt = lax.fori_loop(0, _NCH1, chunk1, jnp.int32(0))
        cnt = jnp.minimum(cnt, jnp.int32(_CAP))
        nch = lax.div(cnt + 15, jnp.int32(16))

        def count_le(vbits):
            vv = zero16i + vbits

            def cc(j, acc):
                sl = pl.ds(j * 16, 16)
                b = plsc.bitcast(cd2_v[sl], jnp.int32)
                ok = ((j * 16 + lane) < cnt) & (b <= vv)
                return acc + jnp.max(plsc.cumsum(ok.astype(jnp.int32)))

            return lax.fori_loop(0, nch, cc, jnp.int32(0))

        def bis(_, lh):
            lo, hi = lh
            mid = lax.div(lo + hi, jnp.int32(2))
            ge = count_le(mid) >= _K
            return (jnp.where(ge, lo, mid + 1), jnp.where(ge, mid, hi))

        _, hi = lax.fori_loop(0, 31, bis,
                              (jnp.int32(0), jnp.int32(_R2BITS)))
        thr = jnp.where(cnt > _K, hi, jnp.int32(_R2BITS))
        thrv = zero16i + thr

        def chunk2(j, base):
            sl = pl.ds(j * 16, 16)
            b = plsc.bitcast(cd2_v[sl], jnp.int32)
            ix = cix_v[sl]
            sel = ((j * 16 + lane) < cnt) & (b <= thrv)
            inc = plsc.cumsum(sel.astype(jnp.int32))
            slot = base + inc - 1
            wm = sel & (slot < _K)
            slotc = jnp.where(wm, i * _K + slot, 0)
            plsc.store_scatter(nbr_v, [slotc], ix, mask=wm)
            plsc.store_scatter(val_v, [slotc], one16f, mask=wm)
            return base + jnp.max(inc)

        lax.fori_loop(0, nch, chunk2, jnp.int32(0))
        return carry

    lax.fori_loop(0, _CPW, per_centroid, 0)
    pltpu.sync_copy(nbr_v, nbr_hbm.at[wid])
    pltpu.sync_copy(val_v, val_hbm.at[wid])


def _ballquery(p16, q16, interpret=False):
    mesh = plsc.VectorSubcoreMesh(core_axis_name="c", subcore_axis_name="s")
    f = pl.kernel(
        _bq_body,
        out_type=(jax.ShapeDtypeStruct((_SC_NW, _SLOTS), jnp.int32),
                  jax.ShapeDtypeStruct((_SC_NW, _SLOTS), jnp.float32)),
        mesh=mesh,
        scratch_types=[pltpu.VMEM((_P16,), jnp.float32)] * 3
        + [pltpu.VMEM((_QPAD,), jnp.float32)] * 3
        + [pltpu.VMEM((_CAP,), jnp.float32), pltpu.VMEM((_CAP,), jnp.int32),
           pltpu.VMEM((_SLOTS,), jnp.int32), pltpu.VMEM((_SLOTS,), jnp.float32)],
        compiler_params=pltpu.CompilerParams(needs_layout_passes=False),
        interpret=interpret,
    )
    return f(p16[0], p16[1], p16[2], q16[0], q16[1], q16[2])


_EPW = _EPAD // _SC_NW
_GCH = 512
_NGC = _EPW // _GCH


def _gather_body(xw_hbm, idx_hbm, xj_hbm, idx_v, xrows_v, sem1):
    wid = lax.axis_index("s") * _SC_NC + lax.axis_index("c")

    def chunk(g, carry):
        base = wid * _EPW + g * _GCH
        pltpu.sync_copy(idx_hbm.at[pl.ds(base, _GCH)], idx_v)
        pltpu.async_copy(xw_hbm.at[idx_v], xrows_v, sem1).wait()
        pltpu.sync_copy(xrows_v, xj_hbm.at[pl.ds(base, _GCH)])
        return carry

    lax.fori_loop(0, _NGC, chunk, 0)


def _scgather(xw, idxflat, interpret=False):
    mesh = plsc.VectorSubcoreMesh(core_axis_name="c", subcore_axis_name="s")
    f = pl.kernel(
        _gather_body,
        out_type=jax.ShapeDtypeStruct((_EPAD, 128), jnp.float32),
        mesh=mesh,
        scratch_types=[pltpu.VMEM((_GCH,), jnp.int32),
                       pltpu.VMEM((_GCH, 128), jnp.float32),
                       pltpu.SemaphoreType.DMA],
        compiler_params=pltpu.CompilerParams(needs_layout_passes=False),
        interpret=interpret,
    )
    return f(xw, idxflat)


def _xw_body(x_ref, w_ref, o_ref):
    o_ref[...] = jnp.dot(x_ref[...], w_ref[...],
                         preferred_element_type=jnp.float32)


def _xw(xpad, w1x, interpret=False):
    return pl.pallas_call(
        _xw_body,
        grid=(10,),
        in_specs=[pl.BlockSpec((1024, 128), lambda i: (i, 0)),
                  pl.BlockSpec((128, 128), lambda i: (0, 0))],
        out_specs=pl.BlockSpec((1024, 128), lambda i: (i, 0)),
        out_shape=jax.ShapeDtypeStruct((_NPAD, 128), jnp.float32),
        interpret=interpret,
    )(xpad, w1x)


def _conv_body(xj_ref, rv_ref, w1r_ref, b1_ref, w2_ref, b2_ref, o_ref):
    xj = xj_ref[...]
    rv = rv_ref[...]
    h1 = xj + jnp.dot(rv, w1r_ref[...],
                      preferred_element_type=jnp.float32) + b1_ref[...]
    h1 = jnp.maximum(h1, 0.0)
    h2 = jnp.dot(h1, w2_ref[...],
                 preferred_element_type=jnp.float32) + b2_ref[...]
    h2 = jnp.maximum(h2, 0.0)
    validc = rv[:, 3:4] > 0.5
    h2m = jnp.where(validc, h2, -jnp.inf)
    mx = jnp.max(h2m.reshape(_BQ, _K, 128), axis=1)
    o_ref[...] = jnp.where(jnp.isfinite(mx), mx, 0.0)


def _conv(xj, relv, w1r8, b1, w2, b2, interpret=False):
    nblk = _MPAD // _BQ
    return pl.pallas_call(
        _conv_body,
        grid=(nblk,),
        in_specs=[pl.BlockSpec((_BQ * _K, 128), lambda c: (c, 0)),
                  pl.BlockSpec((_BQ * _K, 8), lambda c: (c, 0)),
                  pl.BlockSpec((8, 128), lambda c: (0, 0)),
                  pl.BlockSpec((1, 128), lambda c: (0, 0)),
                  pl.BlockSpec((128, 128), lambda c: (0, 0)),
                  pl.BlockSpec((1, 128), lambda c: (0, 0))],
        out_specs=pl.BlockSpec((_BQ, 128), lambda c: (c, 0)),
        out_shape=jax.ShapeDtypeStruct((_MPAD, 128), jnp.float32),
        interpret=interpret,
    )(xj, relv, w1r8, b1.reshape(1, 128), w2, b2.reshape(1, 128))


def kernel(x, pos, batch, Wo1, bo1, g1, be1, Wo2, bo2, g2, be2, Wo3, bo3,
           W1, b1, W2, b2):
    h = pos @ Wo1 + bo1
    h = h / jnp.sqrt(1.0 + _EPS) * g1 + be1
    h = jax.nn.relu(h)
    h = h @ Wo2 + bo2
    h = h / jnp.sqrt(1.0 + _EPS) * g2 + be2
    h = jax.nn.relu(h)
    off = h @ Wo3 + bo3
    dpos = pos + jnp.tanh(off) * 0.1

    padv = jnp.full((_NPAD - _NPTS,), 1e9, jnp.float32)
    px = jnp.concatenate([dpos[:, 0], padv]).reshape(_NROW, 128)
    py = jnp.concatenate([dpos[:, 1], padv]).reshape(_NROW, 128)
    pz = jnp.concatenate([dpos[:, 2], padv]).reshape(_NROW, 128)

    idx = _fps(px, py, pz)
    q = dpos[idx]

    padp = jnp.full((_P16 - _NPTS,), 1e9, jnp.float32)
    p16 = [jnp.concatenate([dpos[:, k], padp]) for k in range(3)]
    padq = jnp.full((_QPAD - _M,), 1e9, jnp.float32)
    q16 = [jnp.concatenate([q[:, k], padq]) for k in range(3)]
    nbr32, val32 = _ballquery(p16, q16)
    nbrp = jnp.pad(nbr32.reshape(_QPAD, _K), ((0, _MPAD - _QPAD), (0, 0)))
    vfp = jnp.pad(val32.reshape(_QPAD, _K), ((0, _MPAD - _QPAD), (0, 0)))

    xpad = jnp.pad(x, ((0, _NPAD - _NPTS), (0, 0)))
    xw = _xw(xpad, W1[:128, :])

    nbrf = nbrp.reshape(-1)
    xj = _scgather(xw, nbrf)

    qp = jnp.pad(q, ((0, _MPAD - _M), (0, 0)))
    relq = dpos[nbrf] - jnp.repeat(qp, _K, axis=0)
    relv = jnp.concatenate(
        [relq, vfp.reshape(-1, 1), jnp.zeros((_EPAD, 4), jnp.float32)],
        axis=1)

    w1r8 = jnp.pad(W1[128:, :], ((0, 5), (0, 0)))

    out = _conv(xj, relv, w1r8, b1, W2, b2)[:_M]
    return (out, q, batch[idx])


# speedup vs baseline: 9.2663x; 1.0192x over previous
import functools

import jax
import jax.numpy as jnp
import numpy as np
from jax import lax
from jax.experimental import pallas as pl
from jax.experimental.pallas import tpu as pltpu
from jax.experimental.pallas import tpu_sc as plsc

_R2 = 0.2 * 0.2
_K = 64
_EPS = 1e-5

_NPTS = 10000
_NROW = 80
_NPAD = _NROW * 128
_M = 2500
_BQ = 8
_MPAD = 2560
_EPAD = _MPAD * _K


def _fps_body(px_ref, py_ref, pz_ref, idx_ref):
    px = px_ref[...]
    py = py_ref[...]
    pz = pz_ref[...]
    rows = lax.broadcasted_iota(jnp.int32, (_NROW, 128), 0)
    cols = lax.broadcasted_iota(jnp.int32, (_NROW, 128), 1)
    iota = rows * 128 + cols
    padm = iota >= _NPTS
    idx_ref[0] = 0

    def dist_to(n):
        msk = iota == n
        x0 = jnp.sum(jnp.where(msk, px, 0.0))
        y0 = jnp.sum(jnp.where(msk, py, 0.0))
        z0 = jnp.sum(jnp.where(msk, pz, 0.0))
        dx = px - x0
        dy = py - y0
        dz = pz - z0
        return dx * dx + dy * dy + dz * dz

    mind = jnp.where(padm, -jnp.inf, dist_to(0))

    def body(i, mind):
        m = jnp.max(mind)
        cand = jnp.where(mind == m, iota, jnp.int32(2**30))
        nxt = jnp.min(cand)
        idx_ref[i] = nxt
        return jnp.minimum(mind, dist_to(nxt))

    lax.fori_loop(1, _M, body, mind)


def _fps(px, py, pz, interpret=False):
    return pl.pallas_call(
        _fps_body,
        out_shape=jax.ShapeDtypeStruct((_M,), jnp.int32),
        out_specs=pl.BlockSpec(memory_space=pltpu.SMEM),
        interpret=interpret,
    )(px, py, pz)


_SC_NC = 2
_SC_NW = 32
_CPW = 79
_QPAD = _SC_NW * _CPW
_P16 = 10016
_NCH1 = _P16 // 16
_CAP = 1024
_SLOTS = _CPW * _K
_R2BITS = int(np.array(_R2, np.float32).view(np.int32))


def _bq_body(px_hbm, py_hbm, pz_hbm, qx_hbm, qy_hbm, qz_hbm,
             nbr_hbm, val_hbm,
             px_v, py_v, pz_v, qx_v, qy_v, qz_v,
             cd2_v, cix_v, nbr_v, val_v):
    wid = lax.axis_index("s") * _SC_NC + lax.axis_index("c")
    pltpu.sync_copy(px_hbm, px_v)
    pltpu.sync_copy(py_hbm, py_v)
    pltpu.sync_copy(pz_hbm, pz_v)
    pltpu.sync_copy(qx_hbm, qx_v)
    pltpu.sync_copy(qy_hbm, qy_v)
    pltpu.sync_copy(qz_hbm, qz_v)

    lane = lax.broadcasted_iota(jnp.int32, (16,), 0)
    zero16i = jnp.zeros((16,), jnp.int32)
    zero16f = jnp.zeros((16,), jnp.float32)
    one16f = jnp.ones((16,), jnp.float32)
    r2v = jnp.full((16,), _R2, jnp.float32)

    def initb(j, carry):
        sl = pl.ds(j * 16, 16)
        nbr_v[sl] = zero16i
        val_v[sl] = zero16f
        return carry

    lax.fori_loop(0, _SLOTS // 16, initb, 0)

    kv = jnp.full((16,), _K, jnp.int32)
    capv = jnp.full((16,), _CAP, jnp.int32)
    r2bv = jnp.full((16,), _R2BITS, jnp.int32)

    def per_centroid(i, carry):
        c = wid * _CPW + i
        cvec = zero16i + c
        qxv = plsc.load_gather(qx_v, [cvec])
        qyv = plsc.load_gather(qy_v, [cvec])
        qzv = plsc.load_gather(qz_v, [cvec])

        def chunk1(j, base_v):
            sl = pl.ds(j * 16, 16)
            dx = px_v[sl] - qxv
            dy = py_v[sl] - qyv
            dz = pz_v[sl] - qzv
            d2 = dx * dx + dy * dy + dz * dz
            msk = d2 <= r2v
            pos = base_v + plsc.cumsum(msk.astype(jnp.int32)) - 1
            wm = msk & (pos < capv)
            posc = jnp.where(wm, pos, 0)
            plsc.store_scatter(cd2_v, [posc], d2, mask=wm)
            plsc.store_scatter(cix_v, [posc], j * 16 + lane, mask=wm)
            return base_v + plsc.all_reduce_population_count(msk)

        cnt_v = lax.fori_loop(0, _NCH1, chunk1, zero16i)
        cnt_v = jnp.minimum(cnt_v, capv)
        cnt = jnp.max(cnt_v)
        nch = lax.div(cnt + 15, jnp.int32(16))

        def bis(_, lh):
            lo_v, hi_v = lh
            mid_v = lax.div(lo_v + hi_v, jnp.int32(2))

            def cc(j, acc_v):
                sl = pl.ds(j * 16, 16)
                b = plsc.bitcast(cd2_v[sl], jnp.int32)
                ok = ((j * 16 + lane) < cnt_v) & (b <= mid_v)
                return acc_v + plsc.all_reduce_population_count(ok)

            cle_v = lax.fori_loop(0, nch, cc, zero16i)
            ge = cle_v >= kv
            return (jnp.where(ge, lo_v, mid_v + 1),
                    jnp.where(ge, mid_v, hi_v))

        _, hi_v = lax.fori_loop(0, 31, bis, (zero16i, r2bv))
        thrv = jnp.where(cnt_v > kv, hi_v, r2bv)

        def chunk2(j, base_v):
            sl = pl.ds(j * 16, 16)
            b = plsc.bitcast(cd2_v[sl], jnp.int32)
            ix = cix_v[sl]
            sel = ((j * 16 + lane) < cnt_v) & (b <= thrv)
            slot = base_v + plsc.cumsum(sel.astype(jnp.int32)) - 1
            wm = sel & (slot < kv)
            slotc = jnp.where(wm, i * _K + slot, 0)
            plsc.store_scatter(nbr_v, [slotc], ix, mask=wm)
            plsc.store_scatter(val_v, [slotc], one16f, mask=wm)
            return base_v + plsc.all_reduce_population_count(sel)

        lax.fori_loop(0, nch, chunk2, zero16i)
        return carry

    lax.fori_loop(0, _CPW, per_centroid, 0)
    pltpu.sync_copy(nbr_v, nbr_hbm.at[wid])
    pltpu.sync_copy(val_v, val_hbm.at[wid])


def _ballquery(p16, q16, interpret=False):
    mesh = plsc.VectorSubcoreMesh(core_axis_name="c", subcore_axis_name="s")
    f = pl.kernel(
        _bq_body,
        out_type=(jax.ShapeDtypeStruct((_SC_NW, _SLOTS), jnp.int32),
                  jax.ShapeDtypeStruct((_SC_NW, _SLOTS), jnp.float32)),
        mesh=mesh,
        scratch_types=[pltpu.VMEM((_P16,), jnp.float32)] * 3
        + [pltpu.VMEM((_QPAD,), jnp.float32)] * 3
        + [pltpu.VMEM((_CAP,), jnp.float32), pltpu.VMEM((_CAP,), jnp.int32),
           pltpu.VMEM((_SLOTS,), jnp.int32), pltpu.VMEM((_SLOTS,), jnp.float32)],
        compiler_params=pltpu.CompilerParams(needs_layout_passes=False),
        interpret=interpret,
    )
    return f(p16[0], p16[1], p16[2], q16[0], q16[1], q16[2])


_EPW = _EPAD // _SC_NW
_GCH = 512
_NGC = _EPW // _GCH


def _gather_body(xw_hbm, idx_hbm, xj_hbm, idx_v, xrows_v, sem1):
    wid = lax.axis_index("s") * _SC_NC + lax.axis_index("c")

    def chunk(g, carry):
        base = wid * _EPW + g * _GCH
        pltpu.sync_copy(idx_hbm.at[pl.ds(base, _GCH)], idx_v)
        pltpu.async_copy(xw_hbm.at[idx_v], xrows_v, sem1).wait()
        pltpu.sync_copy(xrows_v, xj_hbm.at[pl.ds(base, _GCH)])
        return carry

    lax.fori_loop(0, _NGC, chunk, 0)


def _scgather(xw, idxflat, interpret=False):
    mesh = plsc.VectorSubcoreMesh(core_axis_name="c", subcore_axis_name="s")
    f = pl.kernel(
        _gather_body,
        out_type=jax.ShapeDtypeStruct((_EPAD, 128), jnp.float32),
        mesh=mesh,
        scratch_types=[pltpu.VMEM((_GCH,), jnp.int32),
                       pltpu.VMEM((_GCH, 128), jnp.float32),
                       pltpu.SemaphoreType.DMA],
        compiler_params=pltpu.CompilerParams(needs_layout_passes=False),
        interpret=interpret,
    )
    return f(xw, idxflat)


def _xw_body(x_ref, w_ref, o_ref):
    o_ref[...] = jnp.dot(x_ref[...], w_ref[...],
                         preferred_element_type=jnp.float32)


def _xw(xpad, w1x, interpret=False):
    return pl.pallas_call(
        _xw_body,
        grid=(10,),
        in_specs=[pl.BlockSpec((1024, 128), lambda i: (i, 0)),
                  pl.BlockSpec((128, 128), lambda i: (0, 0))],
        out_specs=pl.BlockSpec((1024, 128), lambda i: (i, 0)),
        out_shape=jax.ShapeDtypeStruct((_NPAD, 128), jnp.float32),
        interpret=interpret,
    )(xpad, w1x)


def _conv_body(xj_ref, rv_ref, w1r_ref, b1_ref, w2_ref, b2_ref, o_ref):
    xj = xj_ref[...]
    rv = rv_ref[...]
    h1 = xj + jnp.dot(rv, w1r_ref[...],
                      preferred_element_type=jnp.float32) + b1_ref[...]
    h1 = jnp.maximum(h1, 0.0)
    h2 = jnp.dot(h1, w2_ref[...],
                 preferred_element_type=jnp.float32) + b2_ref[...]
    h2 = jnp.maximum(h2, 0.0)
    validc = rv[:, 3:4] > 0.5
    h2m = jnp.where(validc, h2, -jnp.inf)
    mx = jnp.max(h2m.reshape(_BQ, _K, 128), axis=1)
    o_ref[...] = jnp.where(jnp.isfinite(mx), mx, 0.0)


def _conv(xj, relv, w1r8, b1, w2, b2, interpret=False):
    nblk = _MPAD // _BQ
    return pl.pallas_call(
        _conv_body,
        grid=(nblk,),
        in_specs=[pl.BlockSpec((_BQ * _K, 128), lambda c: (c, 0)),
                  pl.BlockSpec((_BQ * _K, 8), lambda c: (c, 0)),
                  pl.BlockSpec((8, 128), lambda c: (0, 0)),
                  pl.BlockSpec((1, 128), lambda c: (0, 0)),
                  pl.BlockSpec((128, 128), lambda c: (0, 0)),
                  pl.BlockSpec((1, 128), lambda c: (0, 0))],
        out_specs=pl.BlockSpec((_BQ, 128), lambda c: (c, 0)),
        out_shape=jax.ShapeDtypeStruct((_MPAD, 128), jnp.float32),
        interpret=interpret,
    )(xj, relv, w1r8, b1.reshape(1, 128), w2, b2.reshape(1, 128))


def kernel(x, pos, batch, Wo1, bo1, g1, be1, Wo2, bo2, g2, be2, Wo3, bo3,
           W1, b1, W2, b2):
    h = pos @ Wo1 + bo1
    h = h / jnp.sqrt(1.0 + _EPS) * g1 + be1
    h = jax.nn.relu(h)
    h = h @ Wo2 + bo2
    h = h / jnp.sqrt(1.0 + _EPS) * g2 + be2
    h = jax.nn.relu(h)
    off = h @ Wo3 + bo3
    dpos = pos + jnp.tanh(off) * 0.1

    padv = jnp.full((_NPAD - _NPTS,), 1e9, jnp.float32)
    px = jnp.concatenate([dpos[:, 0], padv]).reshape(_NROW, 128)
    py = jnp.concatenate([dpos[:, 1], padv]).reshape(_NROW, 128)
    pz = jnp.concatenate([dpos[:, 2], padv]).reshape(_NROW, 128)

    idx = _fps(px, py, pz)
    q = dpos[idx]

    padp = jnp.full((_P16 - _NPTS,), 1e9, jnp.float32)
    p16 = [jnp.concatenate([dpos[:, k], padp]) for k in range(3)]
    padq = jnp.full((_QPAD - _M,), 1e9, jnp.float32)
    q16 = [jnp.concatenate([q[:, k], padq]) for k in range(3)]
    nbr32, val32 = _ballquery(p16, q16)
    nbrp = jnp.pad(nbr32.reshape(_QPAD, _K), ((0, _MPAD - _QPAD), (0, 0)))
    vfp = jnp.pad(val32.reshape(_QPAD, _K), ((0, _MPAD - _QPAD), (0, 0)))

    xpad = jnp.pad(x, ((0, _NPAD - _NPTS), (0, 0)))
    xw = _xw(xpad, W1[:128, :])

    nbrf = nbrp.reshape(-1)
    xj = _scgather(xw, nbrf)

    qp = jnp.pad(q, ((0, _MPAD - _M), (0, 0)))
    relq = dpos[nbrf] - jnp.repeat(qp, _K, axis=0)
    relv = jnp.concatenate(
        [relq, vfp.reshape(-1, 1), jnp.zeros((_EPAD, 4), jnp.float32)],
        axis=1)

    w1r8 = jnp.pad(W1[128:, :], ((0, 5), (0, 0)))

    out = _conv(xj, relv, w1r8, b1, W2, b2)[:_M]
    return (out, q, batch[idx])


# speedup vs baseline: 9.3081x; 1.0045x over previous
import functools

import jax
import jax.numpy as jnp
import numpy as np
from jax import lax
from jax.experimental import pallas as pl
from jax.experimental.pallas import tpu as pltpu
from jax.experimental.pallas import tpu_sc as plsc

_R2 = 0.2 * 0.2
_K = 64
_EPS = 1e-5

_NPTS = 10000
_NROW = 80
_NPAD = _NROW * 128
_M = 2500
_BQ = 8
_MPAD = 2560
_EPAD = _MPAD * _K


def _fps_body(px_ref, py_ref, pz_ref, idx_ref):
    px = px_ref[...]
    py = py_ref[...]
    pz = pz_ref[...]
    rows = lax.broadcasted_iota(jnp.int32, (_NROW, 128), 0)
    cols = lax.broadcasted_iota(jnp.int32, (_NROW, 128), 1)
    iota = rows * 128 + cols
    padm = iota >= _NPTS
    idx_ref[0] = 0

    lcol = lax.broadcasted_iota(jnp.int32, (1, 128), 1)

    def dist_to(n):
        r = lax.div(n, jnp.int32(128))
        l = lax.rem(n, jnp.int32(128))
        lsel = lcol == l
        x0 = jnp.sum(jnp.where(lsel, px_ref[pl.ds(r, 1), :], 0.0))
        y0 = jnp.sum(jnp.where(lsel, py_ref[pl.ds(r, 1), :], 0.0))
        z0 = jnp.sum(jnp.where(lsel, pz_ref[pl.ds(r, 1), :], 0.0))
        dx = px - x0
        dy = py - y0
        dz = pz - z0
        return dx * dx + dy * dy + dz * dz

    mind = jnp.where(padm, -jnp.inf, dist_to(0))

    def body(i, mind):
        m = jnp.max(mind)
        cand = jnp.where(mind == m, iota, jnp.int32(2**30))
        nxt = jnp.min(cand)
        idx_ref[i] = nxt
        return jnp.minimum(mind, dist_to(nxt))

    lax.fori_loop(1, _M, body, mind)


def _fps(px, py, pz, interpret=False):
    return pl.pallas_call(
        _fps_body,
        out_shape=jax.ShapeDtypeStruct((_M,), jnp.int32),
        out_specs=pl.BlockSpec(memory_space=pltpu.SMEM),
        interpret=interpret,
    )(px, py, pz)


_SC_NC = 2
_SC_NW = 32
_CPW = 79
_QPAD = _SC_NW * _CPW
_P16 = 10016
_NCH1 = _P16 // 16
_CAP = 1024
_SLOTS = _CPW * _K
_R2BITS = int(np.array(_R2, np.float32).view(np.int32))


def _bq_body(px_hbm, py_hbm, pz_hbm, qx_hbm, qy_hbm, qz_hbm,
             nbr_hbm, val_hbm,
             px_v, py_v, pz_v, qx_v, qy_v, qz_v,
             cd2_v, cix_v, nbr_v, val_v):
    wid = lax.axis_index("s") * _SC_NC + lax.axis_index("c")
    pltpu.sync_copy(px_hbm, px_v)
    pltpu.sync_copy(py_hbm, py_v)
    pltpu.sync_copy(pz_hbm, pz_v)
    pltpu.sync_copy(qx_hbm, qx_v)
    pltpu.sync_copy(qy_hbm, qy_v)
    pltpu.sync_copy(qz_hbm, qz_v)

    lane = lax.broadcasted_iota(jnp.int32, (16,), 0)
    zero16i = jnp.zeros((16,), jnp.int32)
    zero16f = jnp.zeros((16,), jnp.float32)
    one16f = jnp.ones((16,), jnp.float32)
    r2v = jnp.full((16,), _R2, jnp.float32)

    def initb(j, carry):
        sl = pl.ds(j * 16, 16)
        nbr_v[sl] = zero16i
        val_v[sl] = zero16f
        return carry

    lax.fori_loop(0, _SLOTS // 16, initb, 0)

    kv = jnp.full((16,), _K, jnp.int32)
    capv = jnp.full((16,), _CAP, jnp.int32)
    r2bv = jnp.full((16,), _R2BITS, jnp.int32)

    def per_centroid(i, carry):
        c = wid * _CPW + i
        cvec = zero16i + c
        qxv = plsc.load_gather(qx_v, [cvec])
        qyv = plsc.load_gather(qy_v, [cvec])
        qzv = plsc.load_gather(qz_v, [cvec])

        def chunk1(j, base_v):
            sl = pl.ds(j * 16, 16)
            dx = px_v[sl] - qxv
            dy = py_v[sl] - qyv
            dz = pz_v[sl] - qzv
            d2 = dx * dx + dy * dy + dz * dz
            msk = d2 <= r2v
            pos = base_v + plsc.cumsum(msk.astype(jnp.int32)) - 1
            wm = msk & (pos < capv)
            posc = jnp.where(wm, pos, 0)
            plsc.store_scatter(cd2_v, [posc], d2, mask=wm)
            plsc.store_scatter(cix_v, [posc], j * 16 + lane, mask=wm)
            return base_v + plsc.all_reduce_population_count(msk)

        cnt_v = lax.fori_loop(0, _NCH1, chunk1, zero16i)
        cnt_v = jnp.minimum(cnt_v, capv)
        cnt = jnp.max(cnt_v)
        nch = lax.div(cnt + 15, jnp.int32(16))

        def bis(_, lh):
            lo_v, hi_v = lh
            mid_v = lax.div(lo_v + hi_v, jnp.int32(2))

            def cc(j, acc_v):
                sl = pl.ds(j * 16, 16)
                b = plsc.bitcast(cd2_v[sl], jnp.int32)
                ok = ((j * 16 + lane) < cnt_v) & (b <= mid_v)
                return acc_v + plsc.all_reduce_population_count(ok)

            cle_v = lax.fori_loop(0, nch, cc, zero16i)
            ge = cle_v >= kv
            return (jnp.where(ge, lo_v, mid_v + 1),
                    jnp.where(ge, mid_v, hi_v))

        _, hi_v = lax.fori_loop(0, 31, bis, (zero16i, r2bv))
        thrv = jnp.where(cnt_v > kv, hi_v, r2bv)

        def chunk2(j, base_v):
            sl = pl.ds(j * 16, 16)
            b = plsc.bitcast(cd2_v[sl], jnp.int32)
            ix = cix_v[sl]
            sel = ((j * 16 + lane) < cnt_v) & (b <= thrv)
            slot = base_v + plsc.cumsum(sel.astype(jnp.int32)) - 1
            wm = sel & (slot < kv)
            slotc = jnp.where(wm, i * _K + slot, 0)
            plsc.store_scatter(nbr_v, [slotc], ix, mask=wm)
            plsc.store_scatter(val_v, [slotc], one16f, mask=wm)
            return base_v + plsc.all_reduce_population_count(sel)

        lax.fori_loop(0, nch, chunk2, zero16i)
        return carry

    lax.fori_loop(0, _CPW, per_centroid, 0)
    pltpu.sync_copy(nbr_v, nbr_hbm.at[wid])
    pltpu.sync_copy(val_v, val_hbm.at[wid])


def _ballquery(p16, q16, interpret=False):
    mesh = plsc.VectorSubcoreMesh(core_axis_name="c", subcore_axis_name="s")
    f = pl.kernel(
        _bq_body,
        out_type=(jax.ShapeDtypeStruct((_SC_NW, _SLOTS), jnp.int32),
                  jax.ShapeDtypeStruct((_SC_NW, _SLOTS), jnp.float32)),
        mesh=mesh,
        scratch_types=[pltpu.VMEM((_P16,), jnp.float32)] * 3
        + [pltpu.VMEM((_QPAD,), jnp.float32)] * 3
        + [pltpu.VMEM((_CAP,), jnp.float32), pltpu.VMEM((_CAP,), jnp.int32),
           pltpu.VMEM((_SLOTS,), jnp.int32), pltpu.VMEM((_SLOTS,), jnp.float32)],
        compiler_params=pltpu.CompilerParams(needs_layout_passes=False),
        interpret=interpret,
    )
    return f(p16[0], p16[1], p16[2], q16[0], q16[1], q16[2])


_EPW = _EPAD // _SC_NW
_GCH = 512
_NGC = _EPW // _GCH


def _gather_body(xw_hbm, idx_hbm, xj_hbm, idx_v, xrows_v, sem1):
    wid = lax.axis_index("s") * _SC_NC + lax.axis_index("c")

    def chunk(g, carry):
        base = wid * _EPW + g * _GCH
        pltpu.sync_copy(idx_hbm.at[pl.ds(base, _GCH)], idx_v)
        pltpu.async_copy(xw_hbm.at[idx_v], xrows_v, sem1).wait()
        pltpu.sync_copy(xrows_v, xj_hbm.at[pl.ds(base, _GCH)])
        return carry

    lax.fori_loop(0, _NGC, chunk, 0)


def _scgather(xw, idxflat, interpret=False):
    mesh = plsc.VectorSubcoreMesh(core_axis_name="c", subcore_axis_name="s")
    f = pl.kernel(
        _gather_body,
        out_type=jax.ShapeDtypeStruct((_EPAD, 128), jnp.float32),
        mesh=mesh,
        scratch_types=[pltpu.VMEM((_GCH,), jnp.int32),
                       pltpu.VMEM((_GCH, 128), jnp.float32),
                       pltpu.SemaphoreType.DMA],
        compiler_params=pltpu.CompilerParams(needs_layout_passes=False),
        interpret=interpret,
    )
    return f(xw, idxflat)


def _xw_body(x_ref, w_ref, o_ref):
    o_ref[...] = jnp.dot(x_ref[...], w_ref[...],
                         preferred_element_type=jnp.float32)


def _xw(xpad, w1x, interpret=False):
    return pl.pallas_call(
        _xw_body,
        grid=(10,),
        in_specs=[pl.BlockSpec((1024, 128), lambda i: (i, 0)),
                  pl.BlockSpec((128, 128), lambda i: (0, 0))],
        out_specs=pl.BlockSpec((1024, 128), lambda i: (i, 0)),
        out_shape=jax.ShapeDtypeStruct((_NPAD, 128), jnp.float32),
        interpret=interpret,
    )(xpad, w1x)


def _conv_body(xj_ref, rv_ref, w1r_ref, b1_ref, w2_ref, b2_ref, o_ref):
    xj = xj_ref[...]
    rv = rv_ref[...]
    h1 = xj + jnp.dot(rv, w1r_ref[...],
                      preferred_element_type=jnp.float32) + b1_ref[...]
    h1 = jnp.maximum(h1, 0.0)
    h2 = jnp.dot(h1, w2_ref[...],
                 preferred_element_type=jnp.float32) + b2_ref[...]
    h2 = jnp.maximum(h2, 0.0)
    validc = rv[:, 3:4] > 0.5
    h2m = jnp.where(validc, h2, -jnp.inf)
    mx = jnp.max(h2m.reshape(_BQ, _K, 128), axis=1)
    o_ref[...] = jnp.where(jnp.isfinite(mx), mx, 0.0)


def _conv(xj, relv, w1r8, b1, w2, b2, interpret=False):
    nblk = _MPAD // _BQ
    return pl.pallas_call(
        _conv_body,
        grid=(nblk,),
        in_specs=[pl.BlockSpec((_BQ * _K, 128), lambda c: (c, 0)),
                  pl.BlockSpec((_BQ * _K, 8), lambda c: (c, 0)),
                  pl.BlockSpec((8, 128), lambda c: (0, 0)),
                  pl.BlockSpec((1, 128), lambda c: (0, 0)),
                  pl.BlockSpec((128, 128), lambda c: (0, 0)),
                  pl.BlockSpec((1, 128), lambda c: (0, 0))],
        out_specs=pl.BlockSpec((_BQ, 128), lambda c: (c, 0)),
        out_shape=jax.ShapeDtypeStruct((_MPAD, 128), jnp.float32),
        interpret=interpret,
    )(xj, relv, w1r8, b1.reshape(1, 128), w2, b2.reshape(1, 128))


def kernel(x, pos, batch, Wo1, bo1, g1, be1, Wo2, bo2, g2, be2, Wo3, bo3,
           W1, b1, W2, b2):
    h = pos @ Wo1 + bo1
    h = h / jnp.sqrt(1.0 + _EPS) * g1 + be1
    h = jax.nn.relu(h)
    h = h @ Wo2 + bo2
    h = h / jnp.sqrt(1.0 + _EPS) * g2 + be2
    h = jax.nn.relu(h)
    off = h @ Wo3 + bo3
    dpos = pos + jnp.tanh(off) * 0.1

    padv = jnp.full((_NPAD - _NPTS,), 1e9, jnp.float32)
    px = jnp.concatenate([dpos[:, 0], padv]).reshape(_NROW, 128)
    py = jnp.concatenate([dpos[:, 1], padv]).reshape(_NROW, 128)
    pz = jnp.concatenate([dpos[:, 2], padv]).reshape(_NROW, 128)

    idx = _fps(px, py, pz)
    q = dpos[idx]

    padp = jnp.full((_P16 - _NPTS,), 1e9, jnp.float32)
    p16 = [jnp.concatenate([dpos[:, k], padp]) for k in range(3)]
    padq = jnp.full((_QPAD - _M,), 1e9, jnp.float32)
    q16 = [jnp.concatenate([q[:, k], padq]) for k in range(3)]
    nbr32, val32 = _ballquery(p16, q16)
    nbrp = jnp.pad(nbr32.reshape(_QPAD, _K), ((0, _MPAD - _QPAD), (0, 0)))
    vfp = jnp.pad(val32.reshape(_QPAD, _K), ((0, _MPAD - _QPAD), (0, 0)))

    xpad = jnp.pad(x, ((0, _NPAD - _NPTS), (0, 0)))
    xw = _xw(xpad, W1[:128, :])

    nbrf = nbrp.reshape(-1)
    xj = _scgather(xw, nbrf)

    qp = jnp.pad(q, ((0, _MPAD - _M), (0, 0)))
    relq = dpos[nbrf] - jnp.repeat(qp, _K, axis=0)
    relv = jnp.concatenate(
        [relq, vfp.reshape(-1, 1), jnp.zeros((_EPAD, 4), jnp.float32)],
        axis=1)

    w1r8 = jnp.pad(W1[128:, :], ((0, 5), (0, 0)))

    out = _conv(xj, relv, w1r8, b1, W2, b2)[:_M]
    return (out, q, batch[idx])
